# Initial kernel scaffold; baseline (speedup 1.0000x reference)
import jax
import jax.numpy as jnp
from jax.experimental import pallas as pl


def kernel(x, codebook):
    raise NotImplementedError("write your pallas kernel here")


# speedup vs baseline: 15.6632x; 15.6632x over previous
import functools

import jax
import jax.numpy as jnp
from jax import lax
from jax.experimental import pallas as pl
from jax.experimental.pallas import tpu as pltpu
from jax.experimental.pallas import tpu_sc as plsc

C = 4
K = 1024
HD = 512
TOK_TILE = 512


def _ids_kernel(x_ref, cb_ref, ids_ref):
    xb = x_ref[...]
    cols = []
    for c in range(C):
        xc = xb[:, c * HD:(c + 1) * HD]
        cb = cb_ref[c]
        comp_sq = jnp.sum(xc * xc, axis=1, keepdims=True)
        cb_sq = jnp.sum(cb * cb, axis=1)[None, :]
        cross = lax.dot_general(
            xc, cb, (((1,), (1,)), ((), ())),
            preferred_element_type=jnp.float32)
        d2 = jnp.maximum((comp_sq + cb_sq) - 2.0 * cross, 0.0)
        dist = jnp.sqrt(d2)
        m = jnp.min(dist, axis=1, keepdims=True)
        iota = lax.broadcasted_iota(jnp.int32, dist.shape, 1)
        idx = jnp.min(jnp.where(dist == m, iota, K), axis=1)
        cols.append((idx + c * K)[:, None])
    ids_ref[...] = jnp.concatenate(cols, axis=1)


def _compute_ids(x2d, codebook):
    n_tok = x2d.shape[0]
    grid = (n_tok // TOK_TILE,)
    return pl.pallas_call(
        _ids_kernel,
        grid=grid,
        in_specs=[
            pl.BlockSpec((TOK_TILE, C * HD), lambda i: (i, 0)),
            pl.BlockSpec((C, K, HD), lambda i: (0, 0, 0)),
        ],
        out_specs=pl.BlockSpec((TOK_TILE, C), lambda i: (i, 0)),
        out_shape=jax.ShapeDtypeStruct((n_tok, C), jnp.int32),
    )(x2d, codebook)


def _make_gather(n_rows):
    info = plsc.get_sparse_core_info()
    nc, ns = info.num_cores, info.num_subcores
    nw = nc * ns
    rows_per_w = n_rows // nw
    chunk = 128
    n_chunks = rows_per_w // chunk
    mesh = plsc.VectorSubcoreMesh(core_axis_name="c", subcore_axis_name="s")

    @functools.partial(
        pl.kernel, mesh=mesh,
        out_type=jax.ShapeDtypeStruct((n_rows, HD), jnp.float32),
        scratch_types=[
            pltpu.VMEM((chunk,), jnp.int32),
            pltpu.VMEM((chunk, HD), jnp.float32),
            pltpu.SemaphoreType.DMA,
        ],
    )
    def gather_k(table_hbm, gid_hbm, out_hbm, idx_v, rows_v, sem):
        wid = lax.axis_index("s") * nc + lax.axis_index("c")
        base = wid * rows_per_w
        for ch in range(n_chunks):
            off = base + ch * chunk
            pltpu.sync_copy(gid_hbm.at[pl.ds(off, chunk)], idx_v)
            pltpu.async_copy(table_hbm.at[idx_v], rows_v, sem).wait()
            pltpu.sync_copy(rows_v, out_hbm.at[pl.ds(off, chunk)])

    return gather_k


def kernel(x, codebook):
    B, S, D = x.shape
    x2d = x.reshape(B * S, D)
    ids = _compute_ids(x2d, codebook)
    gid = ids.reshape(-1)
    table = codebook.reshape(C * K, HD)
    rows = _make_gather(gid.shape[0])(table, gid)
    return rows.reshape(B, S, D)


# speedup vs baseline: 17.8999x; 1.1428x over previous
import functools

import jax
import jax.numpy as jnp
from jax import lax
from jax.experimental import pallas as pl
from jax.experimental.pallas import tpu as pltpu
from jax.experimental.pallas import tpu_sc as plsc

C = 4
K = 1024
HD = 512
TOK_TILE = 512


def _ids_kernel(x_ref, cb_ref, ids_ref, cbsq_ref):
    @pl.when(pl.program_id(0) == 0)
    def _():
        for c in range(C):
            cb = cb_ref[c]
            cbsq_ref[c, :] = jnp.sum(cb * cb, axis=1)

    xb = x_ref[...]
    rows = []
    for c in range(C):
        xc = xb[:, c * HD:(c + 1) * HD]
        cb = cb_ref[c]
        comp_sq = jnp.sum(xc * xc, axis=1, keepdims=True)
        cb_sq = cbsq_ref[c, :][None, :]
        cross = lax.dot_general(
            xc, cb, (((1,), (1,)), ((), ())),
            preferred_element_type=jnp.float32)
        d2 = jnp.maximum((comp_sq + cb_sq) - 2.0 * cross, 0.0)
        dist = jnp.sqrt(d2)
        m = jnp.min(dist, axis=1, keepdims=True)
        iota = lax.broadcasted_iota(jnp.int32, dist.shape, 1)
        idx = jnp.min(jnp.where(dist == m, iota, K), axis=1)
        rows.append((idx + c * K)[None, :])
    ids_ref[...] = jnp.concatenate(rows, axis=0)


def _compute_ids(x2d, codebook):
    n_tok = x2d.shape[0]
    grid = (n_tok // TOK_TILE,)
    return pl.pallas_call(
        _ids_kernel,
        grid=grid,
        in_specs=[
            pl.BlockSpec((TOK_TILE, C * HD), lambda i: (i, 0)),
            pl.BlockSpec((C, K, HD), lambda i: (0, 0, 0)),
        ],
        out_specs=pl.BlockSpec((C, TOK_TILE), lambda i: (0, i)),
        out_shape=jax.ShapeDtypeStruct((C, n_tok), jnp.int32),
        scratch_shapes=[pltpu.VMEM((C, K), jnp.float32)],
    )(x2d, codebook)


def _make_gather(n_tok):
    info = plsc.get_sparse_core_info()
    nc, ns = info.num_cores, info.num_subcores
    nw = nc * ns
    tb = 128
    n_chunks = (n_tok // tb) * C
    per_w = n_chunks // nw
    mesh = plsc.VectorSubcoreMesh(core_axis_name="c", subcore_axis_name="s")

    @functools.partial(
        pl.kernel, mesh=mesh,
        out_type=jax.ShapeDtypeStruct((n_tok, C * HD), jnp.float32),
        scratch_types=[
            pltpu.VMEM((tb,), jnp.int32),
            pltpu.VMEM((tb, HD), jnp.float32),
            pltpu.SemaphoreType.DMA,
        ],
    )
    def gather_k(table_hbm, ids_hbm, out_hbm, idx_v, rows_v, sem):
        wid = lax.axis_index("s") * nc + lax.axis_index("c")
        for j in range(per_w):
            chunk = wid * per_w + j
            c = chunk % C
            t0 = (chunk // C) * tb
            pltpu.sync_copy(ids_hbm.at[c, pl.ds(t0, tb)], idx_v)
            pltpu.async_copy(table_hbm.at[idx_v], rows_v, sem).wait()
            pltpu.sync_copy(rows_v, out_hbm.at[pl.ds(t0, tb), pl.ds(c * HD, HD)])

    return gather_k


def kernel(x, codebook):
    B, S, D = x.shape
    x2d = x.reshape(B * S, D)
    ids = _compute_ids(x2d, codebook)
    table = codebook.reshape(C * K, HD)
    out2d = _make_gather(B * S)(table, ids)
    return out2d.reshape(B, S, D)


# speedup vs baseline: 21.2735x; 1.1885x over previous
import functools

import jax
import jax.numpy as jnp
from jax import lax
from jax.experimental import pallas as pl
from jax.experimental.pallas import tpu as pltpu
from jax.experimental.pallas import tpu_sc as plsc

C = 4
K = 1024
HD = 512
TOK_TILE = 512


def _ids_kernel(x_ref, cb_ref, ids_ref, cbsq_ref):
    @pl.when(pl.program_id(0) == 0)
    def _():
        for c in range(C):
            cb = cb_ref[c]
            cbsq_ref[c, :] = jnp.sum(cb * cb, axis=1)

    xb = x_ref[...]
    rows = []
    for c in range(C):
        xc = xb[:, c * HD:(c + 1) * HD]
        cb = cb_ref[c]
        comp_sq = jnp.sum(xc * xc, axis=1, keepdims=True)
        cb_sq = cbsq_ref[c, :][None, :]
        cross = lax.dot_general(
            xc, cb, (((1,), (1,)), ((), ())),
            preferred_element_type=jnp.float32)
        d2 = jnp.maximum((comp_sq + cb_sq) - 2.0 * cross, 0.0)
        dist = jnp.sqrt(d2)
        m = jnp.min(dist, axis=1, keepdims=True)
        iota = lax.broadcasted_iota(jnp.int32, dist.shape, 1)
        idx = jnp.min(jnp.where(dist == m, iota, K), axis=1)
        rows.append((idx + c * K)[None, :])
    ids_ref[...] = jnp.concatenate(rows, axis=0)


def _compute_ids(x2d, codebook):
    n_tok = x2d.shape[0]
    grid = (n_tok // TOK_TILE,)
    return pl.pallas_call(
        _ids_kernel,
        grid=grid,
        in_specs=[
            pl.BlockSpec((TOK_TILE, C * HD), lambda i: (i, 0)),
            pl.BlockSpec((C, K, HD), lambda i: (0, 0, 0)),
        ],
        out_specs=pl.BlockSpec((C, TOK_TILE), lambda i: (0, i)),
        out_shape=jax.ShapeDtypeStruct((C, n_tok), jnp.int32),
        scratch_shapes=[pltpu.VMEM((C, K), jnp.float32)],
    )(x2d, codebook)


def _make_gather(n_tok):
    info = plsc.get_sparse_core_info()
    nc, ns = info.num_cores, info.num_subcores
    nw = nc * ns
    tb = 32
    n_chunks = n_tok // tb
    per_w = n_chunks // nw
    mesh = plsc.VectorSubcoreMesh(core_axis_name="c", subcore_axis_name="s")

    tok_per_w = n_tok // nw

    @functools.partial(
        pl.kernel, mesh=mesh,
        out_type=jax.ShapeDtypeStruct((n_tok, C * HD), jnp.float32),
        scratch_types=[
            pltpu.VMEM((C, tok_per_w), jnp.int32),
            pltpu.VMEM((tb, C * HD), jnp.float32),
            pltpu.SemaphoreType.DMA,
        ],
    )
    def gather_k(table_hbm, ids_hbm, out_hbm, stage_v, rows_v, sem):
        wid = lax.axis_index("s") * nc + lax.axis_index("c")
        t_base = wid * tok_per_w
        pltpu.sync_copy(ids_hbm.at[:, pl.ds(t_base, tok_per_w)], stage_v)
        for j in range(per_w):
            cps = [pltpu.async_copy(
                table_hbm.at[stage_v.at[c, pl.ds(j * tb, tb)]],
                rows_v.at[:, pl.ds(c * HD, HD)], sem) for c in range(C)]
            for cp in cps:
                cp.wait()
            pltpu.sync_copy(rows_v, out_hbm.at[pl.ds(t_base + j * tb, tb)])

    return gather_k


def kernel(x, codebook):
    B, S, D = x.shape
    x2d = x.reshape(B * S, D)
    ids = _compute_ids(x2d, codebook)
    table = codebook.reshape(C * K, HD)
    out2d = _make_gather(B * S)(table, ids)
    return out2d.reshape(B, S, D)
